# baseline (speedup 1.0000x reference)
import functools

import jax
import jax.numpy as jnp
from jax import lax
from jax.experimental import pallas as pl
from jax.experimental.pallas import tpu as pltpu
from jax.experimental.pallas import tpu_sc as plsc

N = 50000
E = 800000
C = 64

NC = 2
NS = 16
HALF = 25088
ROWS_PER_TILE = HALF // NS
GARBAGE = HALF
ACC_ROWS = HALF + 8

EDGES_PER_TILE = E // NS
CHUNK = 80
GROUP = 5
LAG = 2
G_EDGES = CHUNK * GROUP
NGROUPS = EDGES_PER_TILE // G_EDGES

def _sc_body(x_hbm, src_hbm, tgt_hbm, out_hbm,
             stage_src, stage_tgt, tidx, rowbuf, acc,
             ssem, tsem, isem,
             g0, g1, g2, g3, g4, c0, c1, c2, c3, c4):
    gsems = (g0, g1, g2, g3, g4)
    csems = (c0, c1, c2, c3, c4)
    c = lax.axis_index("c")
    s = lax.axis_index("s")
    base = (c * HALF).astype(jnp.int32)
    vlen = jnp.minimum(jnp.int32(HALF), jnp.int32(N) - base).astype(jnp.uint32)
    tile_rows0 = s * ROWS_PER_TILE
    edge0 = s * EDGES_PER_TILE

    zeros16 = jnp.zeros((16,), jnp.float32)

    def _zrow(r, carry):
        for q in range(C // 16):
            rowbuf[0, r, pl.ds(q * 16, 16)] = zeros16
        return carry

    lax.fori_loop(0, CHUNK, _zrow, 0)
    for j in range(19):
        pltpu.async_copy(rowbuf.at[0],
                         acc.at[pl.ds(tile_rows0 + j * CHUNK, CHUNK)], isem)
    pltpu.async_copy(rowbuf.at[0].at[pl.ds(0, 48)],
                     acc.at[pl.ds(tile_rows0 + 19 * CHUNK, 48)], isem)
    for j in range(19):
        pltpu.make_async_copy(rowbuf.at[0],
                              acc.at[pl.ds(tile_rows0 + j * CHUNK, CHUNK)],
                              isem).wait()
    pltpu.make_async_copy(rowbuf.at[0].at[pl.ds(0, 48)],
                          acc.at[pl.ds(tile_rows0 + 19 * CHUNK, 48)],
                          isem).wait()
    plsc.subcore_barrier()

    pltpu.async_copy(src_hbm.at[pl.ds(edge0, G_EDGES)],
                     stage_src.at[pl.ds(0, G_EDGES)], ssem)
    pltpu.async_copy(tgt_hbm.at[pl.ds(edge0, G_EDGES)],
                     stage_tgt.at[pl.ds(0, G_EDGES)], tsem)

    def _group(G, carry):
        cur = (G % 2) * G_EDGES
        nxt = G_EDGES - cur
        pltpu.make_async_copy(src_hbm.at[pl.ds(edge0, G_EDGES)],
                              stage_src.at[pl.ds(cur, G_EDGES)], ssem).wait()
        pltpu.make_async_copy(tgt_hbm.at[pl.ds(edge0, G_EDGES)],
                              stage_tgt.at[pl.ds(cur, G_EDGES)], tsem).wait()

        @pl.when(G + 1 < NGROUPS)
        def _prefetch():
            off = edge0 + (G + 1) * G_EDGES
            pltpu.async_copy(src_hbm.at[pl.ds(off, G_EDGES)],
                             stage_src.at[pl.ds(nxt, G_EDGES)], ssem)
            pltpu.async_copy(tgt_hbm.at[pl.ds(off, G_EDGES)],
                             stage_tgt.at[pl.ds(nxt, G_EDGES)], tsem)

        def _fire_scatter(j):
            pltpu.make_async_copy(
                x_hbm.at[stage_src.at[pl.ds(cur + j * CHUNK, CHUNK)]],
                rowbuf.at[j], gsems[j]).wait()
            pltpu.async_copy(rowbuf.at[j], acc.at[tidx.at[j]], csems[j],
                             add=True)

        for b in range(GROUP):
            @pl.when(G > 0)
            def _drain_prev(b=b):
                pltpu.make_async_copy(rowbuf.at[b], acc.at[tidx.at[b]],
                                      csems[b]).wait()
            for u in range(CHUNK // 16):
                t = stage_tgt[pl.ds(cur + b * CHUNK + u * 16, 16)]
                lt = t - base
                ok = lt.astype(jnp.uint32) < vlen
                tidx[b, pl.ds(u * 16, 16)] = jnp.where(ok, lt,
                                                       jnp.int32(GARBAGE))
            pltpu.async_copy(
                x_hbm.at[stage_src.at[pl.ds(cur + b * CHUNK, CHUNK)]],
                rowbuf.at[b], gsems[b])
            if b >= LAG:
                _fire_scatter(b - LAG)
            else:
                @pl.when(G > 0)
                def _fire_wrap(b=b):
                    _fire_scatter(b - LAG + GROUP)
        return carry

    lax.fori_loop(0, NGROUPS, _group, 0)
    cur_last = ((NGROUPS - 1) % 2) * G_EDGES
    for b in range(GROUP - LAG, GROUP):
        pltpu.make_async_copy(
            x_hbm.at[stage_src.at[pl.ds(cur_last + b * CHUNK, CHUNK)]],
            rowbuf.at[b], gsems[b]).wait()
        pltpu.async_copy(rowbuf.at[b], acc.at[tidx.at[b]], csems[b],
                         add=True)
    for b in range(GROUP):
        pltpu.make_async_copy(rowbuf.at[b], acc.at[tidx.at[b]],
                              csems[b]).wait()
    plsc.subcore_barrier()
    pltpu.sync_copy(acc.at[pl.ds(tile_rows0, ROWS_PER_TILE)],
                    out_hbm.at[pl.ds(base + tile_rows0, ROWS_PER_TILE)])


_sc_scatter = functools.partial(
    pl.kernel,
    out_type=jax.ShapeDtypeStruct((NC * HALF, C), jnp.float32),
    mesh=plsc.VectorSubcoreMesh(core_axis_name="c", subcore_axis_name="s"),
    scratch_types=[
        pltpu.VMEM((2 * G_EDGES,), jnp.int32),
        pltpu.VMEM((2 * G_EDGES,), jnp.int32),
        pltpu.VMEM((GROUP, CHUNK), jnp.int32),
        pltpu.VMEM((GROUP, CHUNK, C), jnp.float32),
        pltpu.VMEM_SHARED((ACC_ROWS, C), jnp.float32),
        pltpu.SemaphoreType.DMA,
        pltpu.SemaphoreType.DMA,
        pltpu.SemaphoreType.DMA,
    ] + [pltpu.SemaphoreType.DMA] * (2 * GROUP),
    compiler_params=pltpu.CompilerParams(use_tc_tiling_on_sc=False),
)(_sc_body)


BLK = 2000


def _tc_body(x_ref, s_ref, n_ref, w_ref, o_ref):
    h = (x_ref[...] + s_ref[...]) * n_ref[...]
    o_ref[...] = jnp.maximum(
        jnp.dot(h, w_ref[...], preferred_element_type=jnp.float32), 0.0)


_tc_epilogue = pl.pallas_call(
    _tc_body,
    grid=(N // BLK,),
    in_specs=[
        pl.BlockSpec((BLK, C), lambda i: (i, 0)),
        pl.BlockSpec((BLK, C), lambda i: (i, 0)),
        pl.BlockSpec((BLK, 1), lambda i: (i, 0)),
        pl.BlockSpec((C, C), lambda i: (0, 0)),
    ],
    out_specs=pl.BlockSpec((BLK, C), lambda i: (i, 0)),
    out_shape=jax.ShapeDtypeStruct((N, C), jnp.float32),
)


def kernel(x, sources, targets, norm, W):
    src = sources.astype(jnp.int32)
    tgt = targets.astype(jnp.int32)
    scattered = _sc_scatter(x, src, tgt)[:N]
    return _tc_epilogue(x, scattered, norm, W.T)


# speedup vs baseline: 1.0294x; 1.0294x over previous
import functools

import jax
import jax.numpy as jnp
from jax import lax
from jax.experimental import pallas as pl
from jax.experimental.pallas import tpu as pltpu
from jax.experimental.pallas import tpu_sc as plsc

N = 50000
E = 800000
C = 64

NC = 2
NS = 16
HALF = 25088
ROWS_PER_TILE = HALF // NS
GARBAGE = HALF
ACC_ROWS = HALF + 8

EDGES_PER_TILE = E // NS
CHUNK = 80
GROUP = 5
LAG = 2
G_EDGES = CHUNK * GROUP
NGROUPS = EDGES_PER_TILE // G_EDGES

def _sc_body(x_hbm, src_hbm, tgt_hbm, out_hbm,
             stage_src, stage_tgt, tidx, rowbuf, acc,
             ssem, tsem, isem,
             g0, g1, g2, g3, g4, c0, c1, c2, c3, c4):
    gsems = (g0, g1, g2, g3, g4)
    csems = (c0, c1, c2, c3, c4)
    c = lax.axis_index("c")
    s = lax.axis_index("s")
    base = (c * HALF).astype(jnp.int32)
    vlen = jnp.minimum(jnp.int32(HALF), jnp.int32(N) - base).astype(jnp.uint32)
    tile_rows0 = s * ROWS_PER_TILE
    edge0 = s * EDGES_PER_TILE

    zeros16 = jnp.zeros((16,), jnp.float32)

    def _zrow(r, carry):
        for q in range(C // 16):
            rowbuf[0, r, pl.ds(q * 16, 16)] = zeros16
        return carry

    lax.fori_loop(0, CHUNK, _zrow, 0)
    for j in range(19):
        pltpu.async_copy(rowbuf.at[0],
                         acc.at[pl.ds(tile_rows0 + j * CHUNK, CHUNK)], isem)
    pltpu.async_copy(rowbuf.at[0].at[pl.ds(0, 48)],
                     acc.at[pl.ds(tile_rows0 + 19 * CHUNK, 48)], isem)
    for j in range(19):
        pltpu.make_async_copy(rowbuf.at[0],
                              acc.at[pl.ds(tile_rows0 + j * CHUNK, CHUNK)],
                              isem).wait()
    pltpu.make_async_copy(rowbuf.at[0].at[pl.ds(0, 48)],
                          acc.at[pl.ds(tile_rows0 + 19 * CHUNK, 48)],
                          isem).wait()
    plsc.subcore_barrier()

    pltpu.async_copy(src_hbm.at[pl.ds(edge0, G_EDGES)],
                     stage_src.at[pl.ds(0, G_EDGES)], ssem)
    pltpu.async_copy(tgt_hbm.at[pl.ds(edge0, G_EDGES)],
                     stage_tgt.at[pl.ds(0, G_EDGES)], tsem)

    def _group(G, carry):
        cur = (G % 2) * G_EDGES
        nxt = G_EDGES - cur
        pltpu.make_async_copy(src_hbm.at[pl.ds(edge0, G_EDGES)],
                              stage_src.at[pl.ds(cur, G_EDGES)], ssem).wait()
        pltpu.make_async_copy(tgt_hbm.at[pl.ds(edge0, G_EDGES)],
                              stage_tgt.at[pl.ds(cur, G_EDGES)], tsem).wait()

        @pl.when(G + 1 < NGROUPS)
        def _prefetch():
            off = edge0 + (G + 1) * G_EDGES
            pltpu.async_copy(src_hbm.at[pl.ds(off, G_EDGES)],
                             stage_src.at[pl.ds(nxt, G_EDGES)], ssem)
            pltpu.async_copy(tgt_hbm.at[pl.ds(off, G_EDGES)],
                             stage_tgt.at[pl.ds(nxt, G_EDGES)], tsem)

        def _fire_scatter(j):
            pltpu.make_async_copy(
                x_hbm.at[stage_src.at[pl.ds(cur + j * CHUNK, CHUNK)]],
                rowbuf.at[j], gsems[j]).wait()
            pltpu.async_copy(rowbuf.at[j], acc.at[tidx.at[j]], csems[j],
                             add=True)

        for b in range(GROUP):
            @pl.when(G > 0)
            def _drain_prev(b=b):
                pltpu.make_async_copy(rowbuf.at[b], acc.at[tidx.at[b]],
                                      csems[b]).wait()
            for u in range(CHUNK // 16):
                t = stage_tgt[pl.ds(cur + b * CHUNK + u * 16, 16)]
                lt = t - base
                ok = lt.astype(jnp.uint32) < vlen
                tidx[b, pl.ds(u * 16, 16)] = jnp.where(ok, lt,
                                                       jnp.int32(GARBAGE))
            pltpu.async_copy(
                x_hbm.at[stage_src.at[pl.ds(cur + b * CHUNK, CHUNK)]],
                rowbuf.at[b], gsems[b])
            if b >= LAG:
                _fire_scatter(b - LAG)
            else:
                @pl.when(G > 0)
                def _fire_wrap(b=b):
                    _fire_scatter(b - LAG + GROUP)
        return carry

    lax.fori_loop(0, NGROUPS, _group, 0)
    cur_last = ((NGROUPS - 1) % 2) * G_EDGES
    for b in range(GROUP - LAG, GROUP):
        pltpu.make_async_copy(
            x_hbm.at[stage_src.at[pl.ds(cur_last + b * CHUNK, CHUNK)]],
            rowbuf.at[b], gsems[b]).wait()
        pltpu.async_copy(rowbuf.at[b], acc.at[tidx.at[b]], csems[b],
                         add=True)
    for b in range(GROUP):
        pltpu.make_async_copy(rowbuf.at[b], acc.at[tidx.at[b]],
                              csems[b]).wait()
    plsc.subcore_barrier()
    pltpu.sync_copy(acc.at[pl.ds(tile_rows0, ROWS_PER_TILE)],
                    out_hbm.at[pl.ds(base + tile_rows0, ROWS_PER_TILE)])


_sc_scatter = functools.partial(
    pl.kernel,
    out_type=jax.ShapeDtypeStruct((NC * HALF, C), jnp.float32),
    mesh=plsc.VectorSubcoreMesh(core_axis_name="c", subcore_axis_name="s"),
    scratch_types=[
        pltpu.VMEM((2 * G_EDGES,), jnp.int32),
        pltpu.VMEM((2 * G_EDGES,), jnp.int32),
        pltpu.VMEM((GROUP, CHUNK), jnp.int32),
        pltpu.VMEM((GROUP, CHUNK, C), jnp.float32),
        pltpu.VMEM_SHARED((ACC_ROWS, C), jnp.float32),
        pltpu.SemaphoreType.DMA,
        pltpu.SemaphoreType.DMA,
        pltpu.SemaphoreType.DMA,
    ] + [pltpu.SemaphoreType.DMA] * (2 * GROUP),
    compiler_params=pltpu.CompilerParams(use_tc_tiling_on_sc=False),
)(_sc_body)


BLK = 2000


def _tc_body(x_ref, s_ref, n_ref, w_ref, o_ref):
    h = (x_ref[...] + s_ref[...]) * n_ref[...]
    o_ref[...] = jnp.maximum(
        jnp.dot(h, w_ref[...], preferred_element_type=jnp.float32), 0.0)


_tc_epilogue = pl.pallas_call(
    _tc_body,
    grid=(N // BLK,),
    in_specs=[
        pl.BlockSpec((BLK, C), lambda i: (i, 0)),
        pl.BlockSpec((BLK, C), lambda i: (i, 0)),
        pl.BlockSpec((BLK, 1), lambda i: (i, 0)),
        pl.BlockSpec((C, C), lambda i: (0, 0)),
    ],
    out_specs=pl.BlockSpec((BLK, C), lambda i: (i, 0)),
    out_shape=jax.ShapeDtypeStruct((N, C), jnp.float32),
)


def kernel(x, sources, targets, norm, W):
    src = sources.astype(jnp.int32)
    tgt = targets.astype(jnp.int32)
    scattered = _sc_scatter(x, src, tgt)
    return _tc_epilogue(x, scattered, norm, W.T)
